# Initial kernel scaffold; baseline (speedup 1.0000x reference)
import jax
import jax.numpy as jnp
from jax.experimental import pallas as pl


def kernel(img_feats, pcd_feats, img_masks, pcd_masks):
    raise NotImplementedError("write your pallas kernel here")


# speedup vs baseline: 81.1668x; 81.1668x over previous
import jax
import jax.numpy as jnp
from jax import lax
from jax.experimental import pallas as pl

Q = 1008
QP = 1024
K = 4096
D = 256
NCORR = 256
TOPK = 3
NFILL = 512
NCAND = TOPK * QP + NFILL
CH = 512

NEG_FILL = -1e9
NEG_DEAD = -2e9
NEG_RM = -3e9


def _body(img_ref, pcd_ref, imask_ref, pmask_ref, oimg_ref, opcd_ref, osc_ref):
    img = img_ref[...]
    pcd = pcd_ref[...]
    imask = imask_ref[...]
    pmask = pmask_ref[...]

    img_n = img / (jnp.sqrt(jnp.sum(img * img, axis=1, keepdims=True)) + 1e-8)
    pcd_n = pcd / (jnp.sqrt(jnp.sum(pcd * pcd, axis=1, keepdims=True)) + 1e-8)
    s = lax.dot_general(img_n, pcd_n, (((1,), (1,)), ((), ())),
                        preferred_element_type=jnp.float32)

    rowi = lax.broadcasted_iota(jnp.int32, (QP, K), 0)
    coli = lax.broadcasted_iota(jnp.int32, (QP, K), 1)
    realrow = rowi < Q
    valid = (imask > 0.5) & (pmask > 0.5)
    masked = jnp.where(valid & realrow, s,
                       jnp.where(realrow, NEG_FILL, NEG_DEAD))

    workc = masked
    cm = None
    for t in range(TOPK):
        cm = jnp.max(workc, axis=0, keepdims=True)
        if t < TOPK - 1:
            ridx = jnp.min(jnp.where(workc == cm, rowi, QP), axis=0,
                           keepdims=True)
            workc = jnp.where(rowi == ridx, NEG_RM, workc)
    c3 = cm

    okm = (masked >= c3) & (masked > 0.0)

    work = masked
    vals, fids, flags = [], [], []
    for t in range(TOPK):
        m = jnp.max(work, axis=1, keepdims=True)
        idx = jnp.min(jnp.where(work == m, coli, K), axis=1,
                      keepdims=True)
        sel = coli == idx
        fl = jnp.max(jnp.where(sel & okm, 1.0, 0.0), axis=1,
                     keepdims=True) > 0.5
        vals.append(m)
        fids.append(idx)
        flags.append(fl)
        if t < TOPK - 1:
            work = jnp.where(sel, NEG_RM, work)

    qio = rowi[:, :1]
    candv_parts, candf_parts = [], []
    for t in range(TOPK):
        candv_parts.append(jnp.where(flags[t], vals[t], NEG_DEAD))
        candf_parts.append((qio * K + fids[t]).astype(jnp.float32))
    fio = lax.broadcasted_iota(jnp.int32, (NFILL, 1), 0)
    ex = jnp.zeros((NFILL, 1), dtype=jnp.bool_)
    for t in range(TOPK):
        ex = ex | ((fio == fids[t][0:1, 0:1]) & flags[t][0:1, 0:1])
    fv = jnp.where(ex, NEG_DEAD, NEG_FILL)
    candv = jnp.concatenate(candv_parts + [fv], axis=0)
    candf = jnp.concatenate(candf_parts + [fio.astype(jnp.float32)],
                            axis=0)

    ic = lax.broadcasted_iota(jnp.int32, (CH, CH), 0)
    jc = lax.broadcasted_iota(jnp.int32, (CH, CH), 1)
    eye = (ic == jc).astype(jnp.float32)

    def _t(a_col):
        outs = []
        for c in range(a_col.shape[0] // CH):
            seg = lax.dot_general(a_col[c * CH:(c + 1) * CH], eye,
                                  (((0,), (0,)), ((), ())),
                                  preferred_element_type=jnp.float32,
                                  precision=lax.Precision.HIGHEST)
            outs.append(seg)
        return jnp.concatenate(outs, axis=1)

    candv_r = _t(candv)
    candf_r = _t(candf)

    rank = jnp.zeros((NCAND, 1), jnp.float32)
    for c in range(NCAND // CH):
        vj = candv_r[:, c * CH:(c + 1) * CH]
        fj = candf_r[:, c * CH:(c + 1) * CH]
        beats = (vj > candv) | ((vj == candv) & (fj < candf))
        rank += jnp.sum(beats.astype(jnp.float32), axis=1, keepdims=True)
    rank_r = _t(rank)

    rio = lax.broadcasted_iota(jnp.int32, (NCORR, 1), 0).astype(jnp.float32)
    osc = jnp.zeros((NCORR, 1), jnp.float32)
    ofl = jnp.zeros((NCORR, 1), jnp.float32)
    for c in range(NCAND // CH):
        rj = rank_r[:, c * CH:(c + 1) * CH]
        e = rj == rio
        osc += jnp.sum(jnp.where(e, candv_r[:, c * CH:(c + 1) * CH], 0.0),
                       axis=1, keepdims=True)
        ofl += jnp.sum(jnp.where(e, candf_r[:, c * CH:(c + 1) * CH], 0.0),
                       axis=1, keepdims=True)
    fi = ofl.astype(jnp.int32)
    oimg_ref[...] = jnp.right_shift(fi, 12)
    opcd_ref[...] = jnp.bitwise_and(fi, K - 1)
    osc_ref[...] = osc


def kernel(img_feats, pcd_feats, img_masks, pcd_masks):
    img_p = jnp.zeros((QP, D), jnp.float32).at[:Q].set(img_feats)
    imask = jnp.zeros((QP, 1), jnp.float32).at[:Q, 0].set(
        img_masks.astype(jnp.float32))
    pmask = pcd_masks.astype(jnp.float32).reshape(1, K)
    oimg, opcd, osc = pl.pallas_call(
        _body,
        out_shape=(
            jax.ShapeDtypeStruct((NCORR, 1), jnp.int32),
            jax.ShapeDtypeStruct((NCORR, 1), jnp.int32),
            jax.ShapeDtypeStruct((NCORR, 1), jnp.float32),
        ),
    )(img_p, pcd_feats, imask, pmask)
    return oimg.reshape(NCORR), opcd.reshape(NCORR), osc.reshape(NCORR)
